# Initial kernel scaffold; baseline (speedup 1.0000x reference)
import jax
import jax.numpy as jnp
from jax.experimental import pallas as pl


def kernel(x, edge_index, lin_W, lin_b, bn_gamma, bn_beta, mu_W, mu_b, logvar_W, logvar_b):
    raise NotImplementedError("write your pallas kernel here")


# speedup vs baseline: 12.2445x; 12.2445x over previous
import functools

import jax
import jax.numpy as jnp
from jax import lax
from jax.experimental import pallas as pl
from jax.experimental.pallas import tpu as pltpu
from jax.experimental.pallas import tpu_sc as plsc

N = 10000
E = 320000
D = 128
NP = 10240
EPS = 1e-5
CH = 80
NSUB = 16
ROWS_PER_TILE = NP // NSUB

_mesh = plsc.VectorSubcoreMesh(core_axis_name="c", subcore_axis_name="s")


def _edge_targets(rbuf, cbuf, ibuf, base_is_row, c, dummy):
    for j in range(CH // 16):
        sl = pl.ds(j * 16, 16)
        rv = rbuf[sl]
        cv = cbuf[sl]
        base = rv * base_is_row + cv * (1 - base_is_row)
        ibuf[sl] = jnp.where(rv == cv, dummy, base)


@functools.partial(
    pl.kernel,
    mesh=_mesh,
    out_type=jax.ShapeDtypeStruct((2 * NP, D), jnp.float32),
    scratch_types=[
        pltpu.VMEM((CH,), jnp.int32),
        pltpu.VMEM((CH,), jnp.int32),
        pltpu.VMEM((CH,), jnp.int32),
        pltpu.VMEM((CH, D), jnp.float32),
        pltpu.VMEM_SHARED((NP, D), jnp.float32),
    ],
)
def _k1_degrees(row_hbm, col_hbm, zeros_hbm, ones_hbm, out_hbm,
                rbuf, cbuf, ibuf, obuf, acc):
    c = lax.axis_index("c")
    s = lax.axis_index("s")
    r0 = s * ROWS_PER_TILE
    pltpu.sync_copy(zeros_hbm, acc.at[pl.ds(r0, ROWS_PER_TILE)])
    pltpu.sync_copy(ones_hbm, obuf)
    plsc.subcore_barrier()
    e0 = s * (E // NSUB)
    dummy = N + s * 8
    base_is_row = 1 - c

    def body(k, carry):
        base = e0 + k * CH
        pltpu.sync_copy(row_hbm.at[pl.ds(base, CH)], rbuf)
        pltpu.sync_copy(col_hbm.at[pl.ds(base, CH)], cbuf)
        _edge_targets(rbuf, cbuf, ibuf, base_is_row, c, dummy)
        pltpu.sync_copy(obuf, acc.at[ibuf], add=True)
        return carry

    lax.fori_loop(0, (E // NSUB) // CH, body, 0)
    plsc.subcore_barrier()
    pltpu.sync_copy(acc.at[pl.ds(r0, ROWS_PER_TILE)],
                    out_hbm.at[pl.ds(c * NP + r0, ROWS_PER_TILE)])


@functools.partial(
    pl.kernel,
    mesh=_mesh,
    out_type=jax.ShapeDtypeStruct((2 * NP, D), jnp.float32),
    scratch_types=[
        pltpu.VMEM((CH,), jnp.int32),
        pltpu.VMEM((CH,), jnp.int32),
        pltpu.VMEM((CH,), jnp.int32),
        pltpu.VMEM((CH, D), jnp.float32),
        pltpu.VMEM_SHARED((NP, D), jnp.float32),
        pltpu.SemaphoreType.DMA,
    ],
)
def _k2_prop1(xs_hbm, row_hbm, col_hbm, zeros_hbm, out_hbm,
              rbuf, cbuf, ibuf, rows, acc, sem):
    c = lax.axis_index("c")
    s = lax.axis_index("s")
    r0 = s * ROWS_PER_TILE
    pltpu.sync_copy(zeros_hbm, acc.at[pl.ds(r0, ROWS_PER_TILE)])
    plsc.subcore_barrier()
    wid = c * NSUB + s
    e0 = wid * (E // (2 * NSUB))
    dummy = N + s * 8

    def body(k, carry):
        base = e0 + k * CH
        pltpu.sync_copy(row_hbm.at[pl.ds(base, CH)], rbuf)
        pltpu.sync_copy(col_hbm.at[pl.ds(base, CH)], cbuf)
        _edge_targets(rbuf, cbuf, ibuf, 0, c, dummy)
        pltpu.async_copy(xs_hbm.at[rbuf], rows, sem).wait()
        pltpu.sync_copy(rows, acc.at[ibuf], add=True)
        return carry

    lax.fori_loop(0, (E // (2 * NSUB)) // CH, body, 0)
    plsc.subcore_barrier()
    pltpu.sync_copy(acc.at[pl.ds(r0, ROWS_PER_TILE)],
                    out_hbm.at[pl.ds(c * NP + r0, ROWS_PER_TILE)])


@functools.partial(
    pl.kernel,
    mesh=_mesh,
    out_type=jax.ShapeDtypeStruct((2 * NP, D), jnp.float32),
    scratch_types=[
        pltpu.VMEM((CH,), jnp.int32),
        pltpu.VMEM((CH,), jnp.int32),
        pltpu.VMEM((CH,), jnp.int32),
        pltpu.VMEM((CH,), jnp.int32),
        pltpu.VMEM((CH, D), jnp.float32),
        pltpu.VMEM_SHARED((NP, D), jnp.float32),
        pltpu.SemaphoreType.DMA,
    ],
)
def _k4_prop2(ts_hbm, row_hbm, col_hbm, zeros_hbm, out_hbm,
              rbuf, cbuf, ibuf, gbuf, rows, acc, sem):
    c = lax.axis_index("c")
    s = lax.axis_index("s")
    r0 = s * ROWS_PER_TILE
    pltpu.sync_copy(zeros_hbm, acc.at[pl.ds(r0, ROWS_PER_TILE)])
    plsc.subcore_barrier()
    e0 = s * (E // NSUB)
    dummy = N + s * 8
    goff = c * N

    def body(k, carry):
        base = e0 + k * CH
        pltpu.sync_copy(row_hbm.at[pl.ds(base, CH)], rbuf)
        pltpu.sync_copy(col_hbm.at[pl.ds(base, CH)], cbuf)
        _edge_targets(rbuf, cbuf, ibuf, 0, c, dummy)
        for j in range(CH // 16):
            sl = pl.ds(j * 16, 16)
            gbuf[sl] = rbuf[sl] + goff
        pltpu.async_copy(ts_hbm.at[gbuf], rows, sem).wait()
        pltpu.sync_copy(rows, acc.at[ibuf], add=True)
        return carry

    lax.fori_loop(0, (E // NSUB) // CH, body, 0)
    plsc.subcore_barrier()
    pltpu.sync_copy(acc.at[pl.ds(r0, ROWS_PER_TILE)],
                    out_hbm.at[pl.ds(c * NP + r0, ROWS_PER_TILE)])


def _tca_body(degr_ref, degc_ref, x_ref, xs_ref, dr_ref, dc_ref):
    dr = lax.rsqrt(degr_ref[:, 0:1] + 1.0)
    dc = lax.rsqrt(degc_ref[:, 0:1] + 1.0)
    xs_ref[...] = x_ref[...] * dr
    dr_ref[...] = dr
    dc_ref[...] = dc


def _tcb_body(x_ref, s1a_ref, s1b_ref, dr_ref, dc_ref, linw_ref, linb_ref,
              gamma_ref, beta_ref, wcat_ref, ts_ref):
    dr = dr_ref[...]
    h0 = dr * (s1a_ref[...] + s1b_ref[...]) + dr * dr * x_ref[...]
    h1 = jnp.dot(h0, linw_ref[...], preferred_element_type=jnp.float32)
    h1 = h1 + linb_ref[...]
    mean = jnp.mean(h1, axis=0, keepdims=True)
    cent = h1 - mean
    var = jnp.mean(cent * cent, axis=0, keepdims=True)
    h2 = cent * lax.rsqrt(var + EPS) * gamma_ref[...] + beta_ref[...]
    h2 = jnp.maximum(h2, 0.0)
    t = jnp.dot(h2, wcat_ref[...], preferred_element_type=jnp.float32)
    ts = dc_ref[...] * t
    ts_ref[0:N, :] = ts[:, 0:D]
    ts_ref[N:2 * N, :] = ts[:, D:2 * D]


def _tcc_body(s2mu_ref, s2lv_ref, tsmu_ref, tslv_ref, dc_ref, mub_ref,
              lvb_ref, mu_ref, lv_ref):
    dc = dc_ref[...]
    mu_ref[...] = dc * (s2mu_ref[...] + tsmu_ref[...]) + mub_ref[...]
    lv_ref[...] = dc * (s2lv_ref[...] + tslv_ref[...]) + lvb_ref[...]


def kernel(x, edge_index, lin_W, lin_b, bn_gamma, bn_beta, mu_W, mu_b,
           logvar_W, logvar_b):
    f32 = jnp.float32
    row = edge_index[0]
    col = edge_index[1]
    zeros128 = jnp.zeros((ROWS_PER_TILE, D), f32)
    ones128 = jnp.ones((CH, D), f32)

    deg2 = _k1_degrees(row, col, zeros128, ones128)
    degr16 = deg2[0:N]
    degc16 = deg2[NP:NP + N]

    xs, dr, dc = pl.pallas_call(
        _tca_body,
        out_shape=(jax.ShapeDtypeStruct((N, D), f32),
                   jax.ShapeDtypeStruct((N, 1), f32),
                   jax.ShapeDtypeStruct((N, 1), f32)),
    )(degr16, degc16, x)

    s1 = _k2_prop1(xs, row, col, zeros128)

    wcat = jnp.concatenate([mu_W, logvar_W], axis=1)
    ts = pl.pallas_call(
        _tcb_body,
        out_shape=jax.ShapeDtypeStruct((2 * N, D), f32),
    )(x, s1[0:N], s1[NP:NP + N], dr, dc, lin_W, lin_b.reshape(1, -1),
      bn_gamma.reshape(1, -1), bn_beta.reshape(1, -1), wcat)

    s2 = _k4_prop2(ts, row, col, zeros128)

    mu, lv = pl.pallas_call(
        _tcc_body,
        out_shape=(jax.ShapeDtypeStruct((N, D), f32),
                   jax.ShapeDtypeStruct((N, D), f32)),
    )(s2[0:N], s2[NP:NP + N], ts[0:N], ts[N:2 * N], dc,
      mu_b.reshape(1, -1), logvar_b.reshape(1, -1))
    return (mu, lv)


# speedup vs baseline: 28.0655x; 2.2921x over previous
import functools

import jax
import jax.numpy as jnp
from jax import lax
from jax.experimental import pallas as pl
from jax.experimental.pallas import tpu as pltpu
from jax.experimental.pallas import tpu_sc as plsc

N = 10000
E = 320000
D = 128
NP = 10240
EPS = 1e-5
CH = 80
NSUB = 16
ROWS_PER_TILE = NP // NSUB

_mesh = plsc.VectorSubcoreMesh(core_axis_name="c", subcore_axis_name="s")


def _targets16(rv, cv, base_is_row, dummy):
    base = rv * base_is_row + cv * (1 - base_is_row)
    return jnp.where(rv == cv, dummy, base)


def _edge_pipeline(row_hbm, col_hbm, e0, n, acc, dummy, base_is_row,
                   rbufs, cbufs, ibufs, semA, src_for, issue_gather,
                   wait_gather):
    def idx_load(k, b):
        pltpu.async_copy(row_hbm.at[pl.ds(e0 + k * CH, CH)], rbufs[b], semA[b])
        pltpu.async_copy(col_hbm.at[pl.ds(e0 + k * CH, CH)], cbufs[b], semA[b])

    def idx_wait(k, b):
        pltpu.make_async_copy(row_hbm.at[pl.ds(e0 + k * CH, CH)],
                              rbufs[b], semA[b]).wait()
        pltpu.make_async_copy(col_hbm.at[pl.ds(e0 + k * CH, CH)],
                              cbufs[b], semA[b]).wait()

    def compute_ibuf(b):
        for j in range(CH // 16):
            sl = pl.ds(j * 16, 16)
            ibufs[b][sl] = _targets16(rbufs[b][sl], cbufs[b][sl],
                                      base_is_row, dummy)

    idx_load(0, 0)
    idx_load(1, 1)
    idx_wait(0, 0)
    issue_gather(0, 0)

    def slot(k, b):
        compute_ibuf(b)

        @pl.when(k + 1 < n)
        def _():
            idx_wait(k + 1, 1 - b)
            issue_gather(k + 1, 1 - b)
        wait_gather(k, b)

        @pl.when(k + 2 < n)
        def _():
            idx_load(k + 2, b)
        pltpu.sync_copy(src_for(b), acc.at[ibufs[b]], add=True)

    def body(kk, carry):
        k = kk * 2
        slot(k, 0)

        @pl.when(k + 1 < n)
        def _():
            slot(k + 1, 1)
        return carry

    lax.fori_loop(0, (n + 1) // 2, body, 0)


def _writeback(acc, out_hbm, c, s):
    r0 = s * ROWS_PER_TILE
    pltpu.sync_copy(acc.at[pl.ds(r0, ROWS_PER_TILE)],
                    out_hbm.at[pl.ds(c * NP + r0, ROWS_PER_TILE)])


def _zero_acc(zeros_hbm, acc, s):
    pltpu.sync_copy(zeros_hbm, acc.at[pl.ds(s * ROWS_PER_TILE, ROWS_PER_TILE)])


_EPT1 = E // NSUB
_K1 = _EPT1 // CH


@functools.partial(
    pl.kernel,
    mesh=_mesh,
    out_type=jax.ShapeDtypeStruct((2 * NP, D), jnp.float32),
    scratch_types=[
        pltpu.VMEM((CH,), jnp.int32),
        pltpu.VMEM((CH,), jnp.int32),
        pltpu.VMEM((CH,), jnp.int32),
        pltpu.VMEM((CH,), jnp.int32),
        pltpu.VMEM((CH,), jnp.int32),
        pltpu.VMEM((CH,), jnp.int32),
        pltpu.VMEM((CH, D), jnp.float32),
        pltpu.VMEM_SHARED((NP, D), jnp.float32),
        pltpu.SemaphoreType.DMA,
        pltpu.SemaphoreType.DMA,
    ],
)
def _k1_degrees(row_hbm, col_hbm, zeros_hbm, ones_hbm, out_hbm,
                rbuf0, rbuf1, cbuf0, cbuf1, ibuf0, ibuf1, obuf, acc,
                semA0, semA1):
    c = lax.axis_index("c")
    s = lax.axis_index("s")
    _zero_acc(zeros_hbm, acc, s)
    pltpu.sync_copy(ones_hbm, obuf)
    plsc.subcore_barrier()
    _edge_pipeline(
        row_hbm, col_hbm, s * _EPT1, _K1, acc, N + s * 8, 1 - c,
        (rbuf0, rbuf1), (cbuf0, cbuf1), (ibuf0, ibuf1), (semA0, semA1),
        src_for=lambda b: obuf,
        issue_gather=lambda k, b: None,
        wait_gather=lambda k, b: None,
    )
    plsc.subcore_barrier()
    _writeback(acc, out_hbm, c, s)


_EPT2 = E // (2 * NSUB)
_K2 = _EPT2 // CH


@functools.partial(
    pl.kernel,
    mesh=_mesh,
    out_type=jax.ShapeDtypeStruct((2 * NP, D), jnp.float32),
    scratch_types=[
        pltpu.VMEM((CH,), jnp.int32),
        pltpu.VMEM((CH,), jnp.int32),
        pltpu.VMEM((CH,), jnp.int32),
        pltpu.VMEM((CH,), jnp.int32),
        pltpu.VMEM((CH,), jnp.int32),
        pltpu.VMEM((CH,), jnp.int32),
        pltpu.VMEM((CH, D), jnp.float32),
        pltpu.VMEM((CH, D), jnp.float32),
        pltpu.VMEM_SHARED((NP, D), jnp.float32),
        pltpu.SemaphoreType.DMA,
        pltpu.SemaphoreType.DMA,
        pltpu.SemaphoreType.DMA,
        pltpu.SemaphoreType.DMA,
    ],
)
def _k2_prop1(xs_hbm, row_hbm, col_hbm, zeros_hbm, out_hbm,
              rbuf0, rbuf1, cbuf0, cbuf1, ibuf0, ibuf1, rows0, rows1, acc,
              semA0, semA1, semB0, semB1):
    c = lax.axis_index("c")
    s = lax.axis_index("s")
    _zero_acc(zeros_hbm, acc, s)
    plsc.subcore_barrier()
    rbufs = (rbuf0, rbuf1)
    rows = (rows0, rows1)
    semB = (semB0, semB1)

    def issue_gather(k, b):
        pltpu.async_copy(xs_hbm.at[rbufs[b]], rows[b], semB[b])

    def wait_gather(k, b):
        pltpu.make_async_copy(xs_hbm.at[rbufs[b]], rows[b], semB[b]).wait()

    _edge_pipeline(
        row_hbm, col_hbm, (c * NSUB + s) * _EPT2, _K2, acc, N + s * 8, 0,
        rbufs, (cbuf0, cbuf1), (ibuf0, ibuf1), (semA0, semA1),
        src_for=lambda b: rows[b],
        issue_gather=issue_gather,
        wait_gather=wait_gather,
    )
    plsc.subcore_barrier()
    _writeback(acc, out_hbm, c, s)


@functools.partial(
    pl.kernel,
    mesh=_mesh,
    out_type=jax.ShapeDtypeStruct((2 * NP, D), jnp.float32),
    scratch_types=[
        pltpu.VMEM((CH,), jnp.int32),
        pltpu.VMEM((CH,), jnp.int32),
        pltpu.VMEM((CH,), jnp.int32),
        pltpu.VMEM((CH,), jnp.int32),
        pltpu.VMEM((CH,), jnp.int32),
        pltpu.VMEM((CH,), jnp.int32),
        pltpu.VMEM((CH,), jnp.int32),
        pltpu.VMEM((CH,), jnp.int32),
        pltpu.VMEM((CH, D), jnp.float32),
        pltpu.VMEM((CH, D), jnp.float32),
        pltpu.VMEM_SHARED((NP, D), jnp.float32),
        pltpu.SemaphoreType.DMA,
        pltpu.SemaphoreType.DMA,
        pltpu.SemaphoreType.DMA,
        pltpu.SemaphoreType.DMA,
    ],
)
def _k4_prop2(ts_hbm, row_hbm, col_hbm, zeros_hbm, out_hbm,
              rbuf0, rbuf1, cbuf0, cbuf1, ibuf0, ibuf1, gbuf0, gbuf1,
              rows0, rows1, acc, semA0, semA1, semB0, semB1):
    c = lax.axis_index("c")
    s = lax.axis_index("s")
    _zero_acc(zeros_hbm, acc, s)
    plsc.subcore_barrier()
    rbufs = (rbuf0, rbuf1)
    gbufs = (gbuf0, gbuf1)
    rows = (rows0, rows1)
    semB = (semB0, semB1)
    goff = c * N

    def issue_gather(k, b):
        for j in range(CH // 16):
            sl = pl.ds(j * 16, 16)
            gbufs[b][sl] = rbufs[b][sl] + goff
        pltpu.async_copy(ts_hbm.at[gbufs[b]], rows[b], semB[b])

    def wait_gather(k, b):
        pltpu.make_async_copy(ts_hbm.at[gbufs[b]], rows[b], semB[b]).wait()

    _edge_pipeline(
        row_hbm, col_hbm, s * _EPT1, _K1, acc, N + s * 8, 0,
        rbufs, (cbuf0, cbuf1), (ibuf0, ibuf1), (semA0, semA1),
        src_for=lambda b: rows[b],
        issue_gather=issue_gather,
        wait_gather=wait_gather,
    )
    plsc.subcore_barrier()
    _writeback(acc, out_hbm, c, s)


def _tca_body(degr_ref, degc_ref, x_ref, xs_ref, dr_ref, dc_ref):
    dr = lax.rsqrt(degr_ref[:, 0:1] + 1.0)
    dc = lax.rsqrt(degc_ref[:, 0:1] + 1.0)
    xs_ref[...] = x_ref[...] * dr
    dr_ref[...] = dr
    dc_ref[...] = dc


def _tcb_body(x_ref, s1a_ref, s1b_ref, dr_ref, dc_ref, linw_ref, linb_ref,
              gamma_ref, beta_ref, wcat_ref, ts_ref):
    dr = dr_ref[...]
    h0 = dr * (s1a_ref[...] + s1b_ref[...]) + dr * dr * x_ref[...]
    h1 = jnp.dot(h0, linw_ref[...], preferred_element_type=jnp.float32)
    h1 = h1 + linb_ref[...]
    mean = jnp.mean(h1, axis=0, keepdims=True)
    cent = h1 - mean
    var = jnp.mean(cent * cent, axis=0, keepdims=True)
    h2 = cent * lax.rsqrt(var + EPS) * gamma_ref[...] + beta_ref[...]
    h2 = jnp.maximum(h2, 0.0)
    t = jnp.dot(h2, wcat_ref[...], preferred_element_type=jnp.float32)
    ts = dc_ref[...] * t
    ts_ref[0:N, :] = ts[:, 0:D]
    ts_ref[N:2 * N, :] = ts[:, D:2 * D]


def _tcc_body(s2mu_ref, s2lv_ref, tsmu_ref, tslv_ref, dc_ref, mub_ref,
              lvb_ref, mu_ref, lv_ref):
    dc = dc_ref[...]
    mu_ref[...] = dc * (s2mu_ref[...] + tsmu_ref[...]) + mub_ref[...]
    lv_ref[...] = dc * (s2lv_ref[...] + tslv_ref[...]) + lvb_ref[...]


def kernel(x, edge_index, lin_W, lin_b, bn_gamma, bn_beta, mu_W, mu_b,
           logvar_W, logvar_b):
    f32 = jnp.float32
    row = edge_index[0]
    col = edge_index[1]
    zeros128 = jnp.zeros((ROWS_PER_TILE, D), f32)
    ones128 = jnp.ones((CH, D), f32)

    deg2 = _k1_degrees(row, col, zeros128, ones128)
    degr16 = deg2[0:N]
    degc16 = deg2[NP:NP + N]

    xs, dr, dc = pl.pallas_call(
        _tca_body,
        out_shape=(jax.ShapeDtypeStruct((N, D), f32),
                   jax.ShapeDtypeStruct((N, 1), f32),
                   jax.ShapeDtypeStruct((N, 1), f32)),
    )(degr16, degc16, x)

    s1 = _k2_prop1(xs, row, col, zeros128)

    wcat = jnp.concatenate([mu_W, logvar_W], axis=1)
    ts = pl.pallas_call(
        _tcb_body,
        out_shape=jax.ShapeDtypeStruct((2 * N, D), f32),
    )(x, s1[0:N], s1[NP:NP + N], dr, dc, lin_W, lin_b.reshape(1, -1),
      bn_gamma.reshape(1, -1), bn_beta.reshape(1, -1), wcat)

    s2 = _k4_prop2(ts, row, col, zeros128)

    mu, lv = pl.pallas_call(
        _tcc_body,
        out_shape=(jax.ShapeDtypeStruct((N, D), f32),
                   jax.ShapeDtypeStruct((N, D), f32)),
    )(s2[0:N], s2[NP:NP + N], ts[0:N], ts[N:2 * N], dc,
      mu_b.reshape(1, -1), logvar_b.reshape(1, -1))
    return (mu, lv)


# speedup vs baseline: 29.1768x; 1.0396x over previous
import functools

import jax
import jax.numpy as jnp
from jax import lax
from jax.experimental import pallas as pl
from jax.experimental.pallas import tpu as pltpu
from jax.experimental.pallas import tpu_sc as plsc

N = 10000
E = 320000
D = 128
NP = 10240
EPS = 1e-5
CH = 80
NSUB = 16
ROWS_PER_TILE = NP // NSUB

_mesh = plsc.VectorSubcoreMesh(core_axis_name="c", subcore_axis_name="s")


def _targets16(rv, cv, base_is_row, dummy):
    base = rv * base_is_row + cv * (1 - base_is_row)
    return jnp.where(rv == cv, dummy, base)


def _edge_pipeline(row_hbm, col_hbm, e0, n, acc, dummy, base_is_row,
                   rbufs, cbufs, ibufs, semA, src_for, issue_gather,
                   wait_gather):
    def idx_load(k, b):
        pltpu.async_copy(row_hbm.at[pl.ds(e0 + k * CH, CH)], rbufs[b], semA[b])
        pltpu.async_copy(col_hbm.at[pl.ds(e0 + k * CH, CH)], cbufs[b], semA[b])

    def idx_wait(k, b):
        pltpu.make_async_copy(row_hbm.at[pl.ds(e0 + k * CH, CH)],
                              rbufs[b], semA[b]).wait()
        pltpu.make_async_copy(col_hbm.at[pl.ds(e0 + k * CH, CH)],
                              cbufs[b], semA[b]).wait()

    def compute_ibuf(b):
        for j in range(CH // 16):
            sl = pl.ds(j * 16, 16)
            ibufs[b][sl] = _targets16(rbufs[b][sl], cbufs[b][sl],
                                      base_is_row, dummy)

    idx_load(0, 0)
    idx_load(1, 1)
    idx_wait(0, 0)
    issue_gather(0, 0)

    def slot(k, b):
        compute_ibuf(b)

        @pl.when(k + 1 < n)
        def _():
            idx_wait(k + 1, 1 - b)
            issue_gather(k + 1, 1 - b)
        wait_gather(k, b)

        @pl.when(k + 2 < n)
        def _():
            idx_load(k + 2, b)
        pltpu.sync_copy(src_for(b), acc.at[ibufs[b]], add=True)

    def body(kk, carry):
        k = kk * 2
        slot(k, 0)

        @pl.when(k + 1 < n)
        def _():
            slot(k + 1, 1)
        return carry

    lax.fori_loop(0, (n + 1) // 2, body, 0)


def _writeback(acc, out_hbm, c, s):
    r0 = s * ROWS_PER_TILE
    pltpu.sync_copy(acc.at[pl.ds(r0, ROWS_PER_TILE)],
                    out_hbm.at[pl.ds(c * NP + r0, ROWS_PER_TILE)])


def _zero_acc(zeros_hbm, acc, s):
    pltpu.sync_copy(zeros_hbm, acc.at[pl.ds(s * ROWS_PER_TILE, ROWS_PER_TILE)])


_EPT1 = E // NSUB
_K1 = _EPT1 // CH


@functools.partial(
    pl.kernel,
    mesh=_mesh,
    out_type=jax.ShapeDtypeStruct((2 * NP, D), jnp.float32),
    scratch_types=[
        pltpu.VMEM((CH,), jnp.int32),
        pltpu.VMEM((CH,), jnp.int32),
        pltpu.VMEM((CH,), jnp.int32),
        pltpu.VMEM((CH,), jnp.int32),
        pltpu.VMEM((CH,), jnp.int32),
        pltpu.VMEM((CH,), jnp.int32),
        pltpu.VMEM((CH, D), jnp.float32),
        pltpu.VMEM_SHARED((NP, D), jnp.float32),
        pltpu.SemaphoreType.DMA,
        pltpu.SemaphoreType.DMA,
    ],
)
def _k1_degrees(row_hbm, col_hbm, zeros_hbm, ones_hbm, out_hbm,
                rbuf0, rbuf1, cbuf0, cbuf1, ibuf0, ibuf1, obuf, acc,
                semA0, semA1):
    c = lax.axis_index("c")
    s = lax.axis_index("s")
    _zero_acc(zeros_hbm, acc, s)
    pltpu.sync_copy(ones_hbm, obuf)
    plsc.subcore_barrier()
    _edge_pipeline(
        row_hbm, col_hbm, s * _EPT1, _K1, acc, N + s * 8, 1 - c,
        (rbuf0, rbuf1), (cbuf0, cbuf1), (ibuf0, ibuf1), (semA0, semA1),
        src_for=lambda b: obuf,
        issue_gather=lambda k, b: None,
        wait_gather=lambda k, b: None,
    )
    plsc.subcore_barrier()
    _writeback(acc, out_hbm, c, s)


_EPT2 = E // (2 * NSUB)
_K2 = _EPT2 // CH


@functools.partial(
    pl.kernel,
    mesh=_mesh,
    out_type=jax.ShapeDtypeStruct((2 * NP, D), jnp.float32),
    scratch_types=[
        pltpu.VMEM((CH,), jnp.int32),
        pltpu.VMEM((CH,), jnp.int32),
        pltpu.VMEM((CH,), jnp.int32),
        pltpu.VMEM((CH,), jnp.int32),
        pltpu.VMEM((CH,), jnp.int32),
        pltpu.VMEM((CH,), jnp.int32),
        pltpu.VMEM((CH, D), jnp.float32),
        pltpu.VMEM((CH, D), jnp.float32),
        pltpu.VMEM_SHARED((NP, D), jnp.float32),
        pltpu.SemaphoreType.DMA,
        pltpu.SemaphoreType.DMA,
        pltpu.SemaphoreType.DMA,
        pltpu.SemaphoreType.DMA,
    ],
)
def _k2_prop1(xs_hbm, row_hbm, col_hbm, zeros_hbm, out_hbm,
              rbuf0, rbuf1, cbuf0, cbuf1, ibuf0, ibuf1, rows0, rows1, acc,
              semA0, semA1, semB0, semB1):
    c = lax.axis_index("c")
    s = lax.axis_index("s")
    _zero_acc(zeros_hbm, acc, s)
    plsc.subcore_barrier()
    rbufs = (rbuf0, rbuf1)
    rows = (rows0, rows1)
    semB = (semB0, semB1)

    def issue_gather(k, b):
        pltpu.async_copy(xs_hbm.at[rbufs[b]], rows[b], semB[b])

    def wait_gather(k, b):
        pltpu.make_async_copy(xs_hbm.at[rbufs[b]], rows[b], semB[b]).wait()

    _edge_pipeline(
        row_hbm, col_hbm, (c * NSUB + s) * _EPT2, _K2, acc, N + s * 8, 0,
        rbufs, (cbuf0, cbuf1), (ibuf0, ibuf1), (semA0, semA1),
        src_for=lambda b: rows[b],
        issue_gather=issue_gather,
        wait_gather=wait_gather,
    )
    plsc.subcore_barrier()
    _writeback(acc, out_hbm, c, s)


@functools.partial(
    pl.kernel,
    mesh=_mesh,
    out_type=jax.ShapeDtypeStruct((2 * NP, D), jnp.float32),
    scratch_types=[
        pltpu.VMEM((CH,), jnp.int32),
        pltpu.VMEM((CH,), jnp.int32),
        pltpu.VMEM((CH,), jnp.int32),
        pltpu.VMEM((CH,), jnp.int32),
        pltpu.VMEM((CH,), jnp.int32),
        pltpu.VMEM((CH,), jnp.int32),
        pltpu.VMEM((CH,), jnp.int32),
        pltpu.VMEM((CH,), jnp.int32),
        pltpu.VMEM((CH, D), jnp.float32),
        pltpu.VMEM((CH, D), jnp.float32),
        pltpu.VMEM_SHARED((NP, D), jnp.float32),
        pltpu.SemaphoreType.DMA,
        pltpu.SemaphoreType.DMA,
        pltpu.SemaphoreType.DMA,
        pltpu.SemaphoreType.DMA,
    ],
)
def _k4_prop2(ts_hbm, row_hbm, col_hbm, zeros_hbm, out_hbm,
              rbuf0, rbuf1, cbuf0, cbuf1, ibuf0, ibuf1, gbuf0, gbuf1,
              rows0, rows1, acc, semA0, semA1, semB0, semB1):
    c = lax.axis_index("c")
    s = lax.axis_index("s")
    _zero_acc(zeros_hbm, acc, s)
    plsc.subcore_barrier()
    rbufs = (rbuf0, rbuf1)
    gbufs = (gbuf0, gbuf1)
    rows = (rows0, rows1)
    semB = (semB0, semB1)
    goff = c * N

    def issue_gather(k, b):
        for j in range(CH // 16):
            sl = pl.ds(j * 16, 16)
            gbufs[b][sl] = rbufs[b][sl] + goff
        pltpu.async_copy(ts_hbm.at[gbufs[b]], rows[b], semB[b])

    def wait_gather(k, b):
        pltpu.make_async_copy(ts_hbm.at[gbufs[b]], rows[b], semB[b]).wait()

    _edge_pipeline(
        row_hbm, col_hbm, s * _EPT1, _K1, acc, N + s * 8, 0,
        rbufs, (cbuf0, cbuf1), (ibuf0, ibuf1), (semA0, semA1),
        src_for=lambda b: rows[b],
        issue_gather=issue_gather,
        wait_gather=wait_gather,
    )
    plsc.subcore_barrier()
    _writeback(acc, out_hbm, c, s)


def _tca_body(deg_ref, x_ref, xs_ref, dr_ref, dc_ref):
    dr = lax.rsqrt(deg_ref[0:N, 0:1] + 1.0)
    dc = lax.rsqrt(deg_ref[NP:NP + N, 0:1] + 1.0)
    xs_ref[...] = x_ref[...] * dr
    dr_ref[...] = dr
    dc_ref[...] = dc


def _tcb_body(x_ref, s1_ref, dr_ref, dc_ref, linw_ref, linb_ref,
              gamma_ref, beta_ref, wcat_ref, ts_ref):
    dr = dr_ref[...]
    h0 = dr * (s1_ref[0:N] + s1_ref[NP:NP + N]) + dr * dr * x_ref[...]
    h1 = jnp.dot(h0, linw_ref[...], preferred_element_type=jnp.float32)
    h1 = h1 + linb_ref[...]
    mean = jnp.mean(h1, axis=0, keepdims=True)
    cent = h1 - mean
    var = jnp.mean(cent * cent, axis=0, keepdims=True)
    h2 = cent * lax.rsqrt(var + EPS) * gamma_ref[...] + beta_ref[...]
    h2 = jnp.maximum(h2, 0.0)
    t = jnp.dot(h2, wcat_ref[...], preferred_element_type=jnp.float32)
    ts = dc_ref[...] * t
    ts_ref[0:N, :] = ts[:, 0:D]
    ts_ref[N:2 * N, :] = ts[:, D:2 * D]


def _tcc_body(s2_ref, ts_ref, dc_ref, mub_ref, lvb_ref, mu_ref, lv_ref):
    dc = dc_ref[...]
    mu_ref[...] = dc * (s2_ref[0:N] + ts_ref[0:N]) + mub_ref[...]
    lv_ref[...] = dc * (s2_ref[NP:NP + N] + ts_ref[N:2 * N]) + lvb_ref[...]


def kernel(x, edge_index, lin_W, lin_b, bn_gamma, bn_beta, mu_W, mu_b,
           logvar_W, logvar_b):
    f32 = jnp.float32
    row = edge_index[0]
    col = edge_index[1]
    zeros128 = jnp.zeros((ROWS_PER_TILE, D), f32)
    ones128 = jnp.ones((CH, D), f32)

    deg2 = _k1_degrees(row, col, zeros128, ones128)

    xs, dr, dc = pl.pallas_call(
        _tca_body,
        out_shape=(jax.ShapeDtypeStruct((N, D), f32),
                   jax.ShapeDtypeStruct((N, 1), f32),
                   jax.ShapeDtypeStruct((N, 1), f32)),
    )(deg2, x)

    s1 = _k2_prop1(xs, row, col, zeros128)

    wcat = jnp.concatenate([mu_W, logvar_W], axis=1)
    ts = pl.pallas_call(
        _tcb_body,
        out_shape=jax.ShapeDtypeStruct((2 * N, D), f32),
    )(x, s1, dr, dc, lin_W, lin_b.reshape(1, -1),
      bn_gamma.reshape(1, -1), bn_beta.reshape(1, -1), wcat)

    s2 = _k4_prop2(ts, row, col, zeros128)

    mu, lv = pl.pallas_call(
        _tcc_body,
        out_shape=(jax.ShapeDtypeStruct((N, D), f32),
                   jax.ShapeDtypeStruct((N, D), f32)),
    )(s2, ts, dc, mu_b.reshape(1, -1), logvar_b.reshape(1, -1))
    return (mu, lv)
